# baseline (speedup 1.0000x reference)
import jax
import jax.numpy as jnp
from jax.experimental import pallas as pl

_INF = 100000000.0
_PB = 512
_GPAD = 1024


def _fcos_block(data_ref, tab_ref, pts_ref, out_ref):
    xs = pts_ref[:, 0:1]
    ys = pts_ref[:, 1:2]
    ls = pts_ref[:, 2:3]
    us = pts_ref[:, 3:4]

    bx1 = data_ref[0:1, :]
    by1 = data_ref[1:2, :]
    bx2 = data_ref[2:3, :]
    by2 = data_ref[3:4, :]

    l = xs - bx1
    t = ys - by1
    r = bx2 - xs
    b = by2 - ys

    areas = (l + r) * (t + b)
    mind = jnp.minimum(jnp.minimum(l, t), jnp.minimum(r, b))
    maxd = jnp.maximum(jnp.maximum(l, t), jnp.maximum(r, b))
    ok = (mind > 0.0) & (ls <= maxd) & (maxd <= us)
    areas = jnp.where(ok, areas, _INF)

    mv = jnp.min(areas, axis=1, keepdims=True)
    iota = jax.lax.broadcasted_iota(jnp.int32, (_PB, _GPAD), 1)
    idx = jnp.min(jnp.where(areas == mv, iota, _GPAD),
                  axis=1, keepdims=True)
    onehot = jnp.where(iota == idx, 1.0, 0.0).astype(jnp.bfloat16)

    s3 = jax.lax.dot_general(
        onehot, tab_ref[...],
        dimension_numbers=(((1,), (0,)), ((), ())),
        preferred_element_type=jnp.float32)
    sel = (s3[:, 0:8] + s3[:, 8:16]) + s3[:, 16:24]

    l_s = xs - sel[:, 0:1]
    t_s = ys - sel[:, 1:2]
    r_s = sel[:, 2:3] - xs
    b_s = sel[:, 3:4] - ys
    lab_s = sel[:, 4:5]

    zero = jnp.zeros((), jnp.float32)
    cls = jnp.where(mv == _INF, zero, lab_s)
    cnt = jnp.sqrt((jnp.minimum(l_s, t_s) / jnp.maximum(l_s, t_s)) *
                   (jnp.minimum(r_s, b_s) / jnp.maximum(r_s, b_s)))

    out_ref[:, 0:1] = l_s
    out_ref[:, 1:2] = t_s
    out_ref[:, 2:3] = r_s
    out_ref[:, 3:4] = b_s
    out_ref[:, 4:5] = cls
    out_ref[:, 5:6] = cnt
    out_ref[:, 6:8] = jnp.zeros((_PB, 2), jnp.float32)


def kernel(image, bboxes, labels, points, regress_ranges):
    P = points.shape[0]
    G = bboxes.shape[0]
    p_pad = ((P + _PB - 1) // _PB) * _PB

    data = jnp.concatenate(
        [bboxes.T, jnp.zeros((4, G), jnp.float32)], axis=0)
    data = jnp.pad(data, ((0, 0), (0, _GPAD - G)))

    tab_f = jnp.concatenate(
        [bboxes, labels.astype(jnp.float32)[:, None],
         jnp.zeros((G, 3), jnp.float32)], axis=1)
    tab_f = jnp.pad(tab_f, ((0, _GPAD - G), (0, 0)))
    mask = jnp.uint32(0xFFFF0000)
    trunc = lambda v: jax.lax.bitcast_convert_type(
        jax.lax.bitcast_convert_type(v, jnp.uint32) & mask, jnp.float32)
    h1 = trunc(tab_f)
    r1 = tab_f - h1
    h2 = trunc(r1)
    r2 = r1 - h2
    tab = jnp.concatenate(
        [h1.astype(jnp.bfloat16), h2.astype(jnp.bfloat16),
         r2.astype(jnp.bfloat16)], axis=1)

    pts = jnp.concatenate([points, regress_ranges], axis=1)
    pts = jnp.pad(pts, ((0, p_pad - P), (0, 0)))

    out = pl.pallas_call(
        _fcos_block,
        grid=(p_pad // _PB,),
        in_specs=[
            pl.BlockSpec((8, _GPAD), lambda i: (0, 0)),
            pl.BlockSpec((_GPAD, 24), lambda i: (0, 0)),
            pl.BlockSpec((_PB, 4), lambda i: (i, 0)),
        ],
        out_specs=pl.BlockSpec((_PB, 8), lambda i: (i, 0)),
        out_shape=jax.ShapeDtypeStruct((p_pad, 8), jnp.float32),
    )(data, tab, pts)

    reg_targets = out[:P, 0:4]
    cls_targets = out[:P, 4].astype(jnp.int32)
    cnt_targets = out[:P, 5:6]
    return (image, reg_targets, cls_targets, cnt_targets)


# speedup vs baseline: 1.1677x; 1.1677x over previous
import jax
import jax.numpy as jnp
from jax.experimental import pallas as pl
from jax.experimental.pallas import tpu as pltpu

_INF = 100000000.0
_PB = 512


def _fcos_block(bt_ref, bb_ref, lab_ref, pts_ref, rr_ref,
                reg_ref, cls_ref, cnt_ref, tab_scr):
    G = bt_ref.shape[1]

    @pl.when(pl.program_id(0) == 0)
    def _prep():
        labf = lab_ref[...].astype(jnp.float32)
        tf = jnp.concatenate([bb_ref[...], labf], axis=1)
        mask = jnp.uint32(0xFFFF0000)
        trunc = lambda v: jax.lax.bitcast_convert_type(
            jax.lax.bitcast_convert_type(v, jnp.uint32) & mask, jnp.float32)
        h1 = trunc(tf)
        r1 = tf - h1
        h2 = trunc(r1)
        r2 = r1 - h2
        tab_scr[...] = jnp.concatenate(
            [h1.astype(jnp.bfloat16), h2.astype(jnp.bfloat16),
             r2.astype(jnp.bfloat16)], axis=1)

    xs = pts_ref[:, 0:1]
    ys = pts_ref[:, 1:2]
    ls = rr_ref[:, 0:1]
    us = rr_ref[:, 1:2]

    bx1 = bt_ref[0:1, :]
    by1 = bt_ref[1:2, :]
    bx2 = bt_ref[2:3, :]
    by2 = bt_ref[3:4, :]

    l = xs - bx1
    t = ys - by1
    r = bx2 - xs
    b = by2 - ys

    areas = (l + r) * (t + b)
    mind = jnp.minimum(jnp.minimum(l, t), jnp.minimum(r, b))
    maxd = jnp.maximum(jnp.maximum(l, t), jnp.maximum(r, b))
    ok = (mind > 0.0) & (ls <= maxd) & (maxd <= us)
    areas = jnp.where(ok, areas, _INF)

    mv = jnp.min(areas, axis=1, keepdims=True)
    iota = jax.lax.broadcasted_iota(jnp.int32, (_PB, G), 1)
    idx = jnp.min(jnp.where(areas == mv, iota, G),
                  axis=1, keepdims=True)
    onehot = jnp.where(iota == idx, 1.0, 0.0).astype(jnp.bfloat16)

    s3 = jax.lax.dot_general(
        onehot, tab_scr[...],
        dimension_numbers=(((1,), (0,)), ((), ())),
        preferred_element_type=jnp.float32)
    sel = (s3[:, 0:5] + s3[:, 5:10]) + s3[:, 10:15]

    l_s = xs - sel[:, 0:1]
    t_s = ys - sel[:, 1:2]
    r_s = sel[:, 2:3] - xs
    b_s = sel[:, 3:4] - ys
    lab_s = sel[:, 4:5]

    cls = jnp.where(mv == _INF, 0, lab_s.astype(jnp.int32))
    cnt = jnp.sqrt((jnp.minimum(l_s, t_s) / jnp.maximum(l_s, t_s)) *
                   (jnp.minimum(r_s, b_s) / jnp.maximum(r_s, b_s)))

    reg_ref[:, 0:1] = l_s
    reg_ref[:, 1:2] = t_s
    reg_ref[:, 2:3] = r_s
    reg_ref[:, 3:4] = b_s
    cls_ref[...] = cls
    cnt_ref[...] = cnt


def kernel(image, bboxes, labels, points, regress_ranges):
    P = points.shape[0]
    G = bboxes.shape[0]
    nblk = (P + _PB - 1) // _PB

    bboxes_t = bboxes.T
    labels2 = labels[:, None]

    reg, cls2, cnt = pl.pallas_call(
        _fcos_block,
        grid=(nblk,),
        in_specs=[
            pl.BlockSpec((4, G), lambda i: (0, 0)),
            pl.BlockSpec((G, 4), lambda i: (0, 0)),
            pl.BlockSpec((G, 1), lambda i: (0, 0)),
            pl.BlockSpec((_PB, 2), lambda i: (i, 0)),
            pl.BlockSpec((_PB, 2), lambda i: (i, 0)),
        ],
        out_specs=[
            pl.BlockSpec((_PB, 4), lambda i: (i, 0)),
            pl.BlockSpec((_PB, 1), lambda i: (i, 0)),
            pl.BlockSpec((_PB, 1), lambda i: (i, 0)),
        ],
        out_shape=[
            jax.ShapeDtypeStruct((P, 4), jnp.float32),
            jax.ShapeDtypeStruct((P, 1), jnp.int32),
            jax.ShapeDtypeStruct((P, 1), jnp.float32),
        ],
        scratch_shapes=[pltpu.VMEM((G, 15), jnp.bfloat16)],
    )(bboxes_t, bboxes, labels2, points, regress_ranges)

    return (image, reg, cls2[:, 0], cnt)
